# speedup vs baseline: 6.3325x; 6.3325x over previous
import jax
import jax.numpy as jnp
from jax import lax
from jax.experimental import pallas as pl
from jax.experimental.pallas import tpu as pltpu
from jax.experimental.pallas import tpu_sc as plsc

NC, NS = 2, 16
NW = NC * NS
CHUNK = 128


def _gather_body(ids_hbm, table_hbm, out_hbm, idx_v, rows_v, gsem):
    wid = lax.axis_index("s") * NC + lax.axis_index("c")
    per_w = ids_hbm.shape[0] // NW
    steps = per_w // CHUNK
    base = wid * per_w
    pltpu.sync_copy(ids_hbm.at[pl.ds(base, per_w)], idx_v)

    def step(i, carry):
        off = pl.multiple_of(i * CHUNK, CHUNK)
        pltpu.async_copy(
            table_hbm.at[idx_v.at[pl.ds(off, CHUNK)]], rows_v, gsem
        ).wait()
        pltpu.sync_copy(rows_v, out_hbm.at[pl.ds(base + off, CHUNK)])
        return carry

    lax.fori_loop(0, steps, step, 0)


def kernel(input_ids, word_embeddings):
    B, L = input_ids.shape
    V, D = word_embeddings.shape
    total = B * L
    ids = input_ids.reshape(total).astype(jnp.int32)
    per_w = total // NW

    mesh = plsc.VectorSubcoreMesh(core_axis_name="c", subcore_axis_name="s")
    k = pl.kernel(
        _gather_body,
        mesh=mesh,
        out_type=jax.ShapeDtypeStruct((total, D), jnp.float32),
        scratch_types=[
            pltpu.VMEM((per_w,), jnp.int32),
            pltpu.VMEM((CHUNK, D), jnp.float32),
            pltpu.SemaphoreType.DMA,
        ],
    )
    out = k(ids, word_embeddings)
    return out.reshape(B, L, D)


# speedup vs baseline: 9.2612x; 1.4625x over previous
import jax
import jax.numpy as jnp
from jax import lax
from jax.experimental import pallas as pl
from jax.experimental.pallas import tpu as pltpu
from jax.experimental.pallas import tpu_sc as plsc

NC, NS = 2, 16
NW = NC * NS
CHUNK = 128
NBUF = 4


def _gather_body(ids_hbm, table_hbm, out_hbm, idx_v, rows_v, *sems):
    gsem, wsem = sems[:NBUF], sems[NBUF:]
    wid = lax.axis_index("s") * NC + lax.axis_index("c")
    per_w = ids_hbm.shape[0] // NW
    nout = per_w // (CHUNK * NBUF)
    base = wid * per_w
    pltpu.sync_copy(ids_hbm.at[pl.ds(base, per_w)], idx_v)

    def g_copy(i, b):
        off = pl.multiple_of(i * CHUNK, CHUNK)
        return pltpu.make_async_copy(
            table_hbm.at[idx_v.at[pl.ds(off, CHUNK)]], rows_v.at[b], gsem[b]
        )

    def w_copy(i, b):
        off = pl.multiple_of(i * CHUNK, CHUNK)
        return pltpu.make_async_copy(
            rows_v.at[b], out_hbm.at[pl.ds(base + off, CHUNK)], wsem[b]
        )

    for b in range(NBUF):
        g_copy(b, b).start()

    def outer(o, carry):
        for b in range(NBUF):
            i = o * NBUF + b
            g_copy(i, b).wait()
            w_copy(i, b).start()
            w_copy(i, b).wait()
            g_copy(i + NBUF, b).start()
        return carry

    lax.fori_loop(0, nout - 1, outer, 0)

    for b in range(NBUF):
        i = (nout - 1) * NBUF + b
        g_copy(i, b).wait()
        w_copy(i, b).start()
    for b in range(NBUF):
        w_copy((nout - 1) * NBUF + b, b).wait()


def kernel(input_ids, word_embeddings):
    B, L = input_ids.shape
    V, D = word_embeddings.shape
    total = B * L
    ids = input_ids.reshape(total).astype(jnp.int32)
    per_w = total // NW

    mesh = plsc.VectorSubcoreMesh(core_axis_name="c", subcore_axis_name="s")
    k = pl.kernel(
        _gather_body,
        mesh=mesh,
        out_type=jax.ShapeDtypeStruct((total, D), jnp.float32),
        scratch_types=[
            pltpu.VMEM((per_w,), jnp.int32),
            pltpu.VMEM((NBUF, CHUNK, D), jnp.float32),
        ] + [pltpu.SemaphoreType.DMA] * (2 * NBUF),
    )
    out = k(ids, word_embeddings)
    return out.reshape(B, L, D)
